# baseline (speedup 1.0000x reference)
import functools

import jax
import jax.numpy as jnp
from jax import lax
from jax.experimental import pallas as pl
from jax.experimental.pallas import tpu as pltpu
from jax.experimental.pallas import tpu_sc as plsc

M_BINS = 65536
LANES = 16
SLAB = 8
CCOLS = 1024


def _sc_body(n_keys, n_rows, n_cols,
             alpha_hbm, start_hbm, aext_hbm, s_hbm, c_hbm, scal_hbm,
             out_hbm,
             start_v, aext_v, s_v, c_v, scal_v,
             in0_v, in1_v, out0_v, out1_v,
             sem_in0, sem_in1, sem_out0, sem_out1):
    nc = 2
    wid = lax.axis_index("s") * nc + lax.axis_index("c")
    rows_per_w = n_rows // 32
    base_row = wid * rows_per_w
    col_chunks = n_cols // CCOLS
    n_chunks = (rows_per_w // SLAB) * col_chunks
    in_bufs = (in0_v, in1_v)
    out_bufs = (out0_v, out1_v)
    sem_in = (sem_in0, sem_in1)
    sem_out = (sem_out0, sem_out1)

    def chunk_off(t):
        r0 = base_row + (t // col_chunks) * SLAB
        c0 = (t % col_chunks) * CCOLS
        return r0, c0

    pltpu.sync_copy(start_hbm, start_v)
    pltpu.sync_copy(aext_hbm, aext_v)
    pltpu.sync_copy(s_hbm, s_v)
    pltpu.sync_copy(c_hbm, c_v)
    pltpu.sync_copy(scal_hbm, scal_v)

    a0v = scal_v[0, :]
    a1v = scal_v[1, :]
    invwv = scal_v[2, :]
    b0v = scal_v[3, :]

    for b in range(2):
        r0, c0 = chunk_off(b)
        pltpu.async_copy(alpha_hbm.at[pl.ds(r0, SLAB), pl.ds(c0, CCOLS)],
                         in_bufs[b], sem_in[b])

    def compute(in_v, out_v):
        @plsc.parallel_loop(0, CCOLS, LANES, unroll=4)
        def vreg_body(o):
            for r in range(SLAB):
                v = in_v[r, pl.ds(o, LANES)]
                neg = v < 0.0
                vc = jnp.minimum(jnp.maximum(v, a0v), a1v)
                u = (vc - a0v) * invwv
                m = jnp.minimum(u.astype(jnp.int32), M_BINS - 1)
                j = plsc.load_gather(start_v, [m], mask=neg)
                g = plsc.load_gather(aext_v, [jnp.clip(j, 0, n_keys)],
                                     mask=neg)
                j = j + jnp.where(g <= vc, 1, 0)
                seg = jnp.clip(j, 1, n_keys - 1) - 1
                sv = plsc.load_gather(s_v, [seg], mask=neg)
                cv = plsc.load_gather(c_v, [seg], mask=neg)
                res = sv * vc + cv
                res = jnp.where(neg, res, b0v)
                out_v[r, pl.ds(o, LANES)] = res

    def pair_body(ii, _):
        for b in range(2):
            t = ii * 2 + b
            r0, c0 = chunk_off(t)
            pltpu.make_async_copy(
                alpha_hbm.at[pl.ds(r0, SLAB), pl.ds(c0, CCOLS)],
                in_bufs[b], sem_in[b]).wait()
            @pl.when(ii > 0)
            def _wait_out():
                pltpu.make_async_copy(
                    out_bufs[b],
                    out_hbm.at[pl.ds(r0, SLAB), pl.ds(c0, CCOLS)],
                    sem_out[b]).wait()
            compute(in_bufs[b], out_bufs[b])
            pltpu.async_copy(out_bufs[b],
                             out_hbm.at[pl.ds(r0, SLAB), pl.ds(c0, CCOLS)],
                             sem_out[b])
            @pl.when(t + 2 < n_chunks)
            def _prefetch():
                r2, c2 = chunk_off(t + 2)
                pltpu.async_copy(
                    alpha_hbm.at[pl.ds(r2, SLAB), pl.ds(c2, CCOLS)],
                    in_bufs[b], sem_in[b])
        return 0

    lax.fori_loop(0, n_chunks // 2, pair_body, 0)
    for b in range(2):
        r0, c0 = chunk_off(0)
        pltpu.make_async_copy(out_bufs[b],
                              out_hbm.at[pl.ds(r0, SLAB), pl.ds(c0, CCOLS)],
                              sem_out[b]).wait()


def kernel(alpha, alpha_tab, beta_tab):
    n = alpha_tab.shape[0]
    n_rows, n_cols = alpha.shape

    a0 = alpha_tab[0]
    a1 = alpha_tab[-1]
    inv_w = jnp.float32(M_BINS) / (a1 - a0)

    mk = jnp.minimum(((alpha_tab - a0) * inv_w).astype(jnp.int32), M_BINS - 1)
    counts = jnp.zeros((M_BINS,), jnp.int32).at[mk].add(1)
    start = (jnp.cumsum(counts) - counts).astype(jnp.int32)

    tab_pad = (-n) % 8 + 8
    aext = jnp.concatenate(
        [alpha_tab, jnp.full((tab_pad,), jnp.inf, jnp.float32)])
    s_seg = (beta_tab[1:] - beta_tab[:-1]) / (
        alpha_tab[1:] - alpha_tab[:-1] + 1e-12)
    c_seg = beta_tab[:-1] - s_seg * alpha_tab[:-1]
    pad1 = jnp.zeros(((-(n - 1)) % 8 + 8,), jnp.float32)
    s_seg = jnp.concatenate([s_seg, pad1])
    c_seg = jnp.concatenate([c_seg, pad1])

    scal = jnp.stack([
        jnp.full((LANES,), a0, jnp.float32),
        jnp.full((LANES,), a1, jnp.float32),
        jnp.full((LANES,), inv_w, jnp.float32),
        jnp.full((LANES,), beta_tab[-1], jnp.float32),
    ])

    tab_len = n + tab_pad
    seg_len = (n - 1) + (-(n - 1)) % 8 + 8
    mesh = plsc.VectorSubcoreMesh(core_axis_name="c", subcore_axis_name="s")
    run = pl.kernel(
        functools.partial(_sc_body, n, n_rows, n_cols),
        out_type=jax.ShapeDtypeStruct((n_rows, n_cols), jnp.float32),
        mesh=mesh,
        scratch_types=[
            pltpu.VMEM((M_BINS,), jnp.int32),
            pltpu.VMEM((tab_len,), jnp.float32),
            pltpu.VMEM((seg_len,), jnp.float32),
            pltpu.VMEM((seg_len,), jnp.float32),
            pltpu.VMEM((4, LANES), jnp.float32),
            pltpu.VMEM((SLAB, CCOLS), jnp.float32),
            pltpu.VMEM((SLAB, CCOLS), jnp.float32),
            pltpu.VMEM((SLAB, CCOLS), jnp.float32),
            pltpu.VMEM((SLAB, CCOLS), jnp.float32),
            pltpu.SemaphoreType.DMA,
            pltpu.SemaphoreType.DMA,
            pltpu.SemaphoreType.DMA,
            pltpu.SemaphoreType.DMA,
        ],
        compiler_params=pltpu.CompilerParams(
            needs_layout_passes=False, use_tc_tiling_on_sc=True),
    )
    return run(alpha, start, aext, s_seg, c_seg, scal)


# speedup vs baseline: 1.4028x; 1.4028x over previous
import functools

import jax
import jax.numpy as jnp
from jax import lax
from jax.experimental import pallas as pl
from jax.experimental.pallas import tpu as pltpu
from jax.experimental.pallas import tpu_sc as plsc

M_BINS = 65536
LANES = 16
SLAB = 8
CCOLS = 1024


def _sc_body(n_keys, n_rows, n_cols,
             alpha_hbm, start_hbm, aext_hbm, s_hbm, c_hbm, scal_hbm,
             out_hbm,
             start_v, aext_v, s_v, c_v, scal_v,
             in0_v, in1_v, out0_v, out1_v,
             sem_in0, sem_in1, sem_out0, sem_out1):
    nc = 2
    wid = lax.axis_index("s") * nc + lax.axis_index("c")
    rows_per_w = n_rows // 32
    base_row = wid * rows_per_w
    col_chunks = n_cols // CCOLS
    n_chunks = (rows_per_w // SLAB) * col_chunks
    in_bufs = (in0_v, in1_v)
    out_bufs = (out0_v, out1_v)
    sem_in = (sem_in0, sem_in1)
    sem_out = (sem_out0, sem_out1)

    def chunk_off(t):
        r0 = base_row + (t // col_chunks) * SLAB
        c0 = (t % col_chunks) * CCOLS
        return r0, c0

    pltpu.sync_copy(start_hbm, start_v)
    pltpu.sync_copy(aext_hbm, aext_v)
    pltpu.sync_copy(s_hbm, s_v)
    pltpu.sync_copy(c_hbm, c_v)
    pltpu.sync_copy(scal_hbm, scal_v)

    a0v = scal_v[0, :]
    a1v = scal_v[1, :]
    invwv = scal_v[2, :]
    b0v = scal_v[3, :]

    for b in range(2):
        r0, c0 = chunk_off(b)
        pltpu.async_copy(alpha_hbm.at[pl.ds(r0, SLAB), pl.ds(c0, CCOLS)],
                         in_bufs[b], sem_in[b])

    def compute(in_v, out_v):
        @plsc.parallel_loop(0, CCOLS, LANES, unroll=1)
        def vreg_body(o):
            for r in range(SLAB):
                v = in_v[r, pl.ds(o, LANES)]
                neg = v < 0.0
                vc = jnp.minimum(jnp.maximum(v, a0v), a1v)
                u = (vc - a0v) * invwv
                m = jnp.minimum(u.astype(jnp.int32), M_BINS - 1)
                j = plsc.load_gather(start_v, [m], mask=neg)
                g = plsc.load_gather(aext_v, [jnp.clip(j, 0, n_keys)],
                                     mask=neg)
                j = j + jnp.where(g <= vc, 1, 0)
                seg = jnp.clip(j, 1, n_keys - 1) - 1
                sv = plsc.load_gather(s_v, [seg], mask=neg)
                cv = plsc.load_gather(c_v, [seg], mask=neg)
                res = sv * vc + cv
                res = jnp.where(neg, res, b0v)
                out_v[r, pl.ds(o, LANES)] = res

    def pair_body(ii, _):
        for b in range(2):
            t = ii * 2 + b
            r0, c0 = chunk_off(t)
            pltpu.make_async_copy(
                alpha_hbm.at[pl.ds(r0, SLAB), pl.ds(c0, CCOLS)],
                in_bufs[b], sem_in[b]).wait()
            @pl.when(ii > 0)
            def _wait_out():
                pltpu.make_async_copy(
                    out_bufs[b],
                    out_hbm.at[pl.ds(r0, SLAB), pl.ds(c0, CCOLS)],
                    sem_out[b]).wait()
            compute(in_bufs[b], out_bufs[b])
            pltpu.async_copy(out_bufs[b],
                             out_hbm.at[pl.ds(r0, SLAB), pl.ds(c0, CCOLS)],
                             sem_out[b])
            @pl.when(t + 2 < n_chunks)
            def _prefetch():
                r2, c2 = chunk_off(t + 2)
                pltpu.async_copy(
                    alpha_hbm.at[pl.ds(r2, SLAB), pl.ds(c2, CCOLS)],
                    in_bufs[b], sem_in[b])
        return 0

    lax.fori_loop(0, n_chunks // 2, pair_body, 0)
    for b in range(2):
        r0, c0 = chunk_off(0)
        pltpu.make_async_copy(out_bufs[b],
                              out_hbm.at[pl.ds(r0, SLAB), pl.ds(c0, CCOLS)],
                              sem_out[b]).wait()


def kernel(alpha, alpha_tab, beta_tab):
    n = alpha_tab.shape[0]
    n_rows, n_cols = alpha.shape

    a0 = alpha_tab[0]
    a1 = alpha_tab[-1]
    inv_w = jnp.float32(M_BINS) / (a1 - a0)

    mk = jnp.minimum(((alpha_tab - a0) * inv_w).astype(jnp.int32), M_BINS - 1)
    counts = jnp.zeros((M_BINS,), jnp.int32).at[mk].add(1)
    start = (jnp.cumsum(counts) - counts).astype(jnp.int32)

    tab_pad = (-n) % 8 + 8
    aext = jnp.concatenate(
        [alpha_tab, jnp.full((tab_pad,), jnp.inf, jnp.float32)])
    s_seg = (beta_tab[1:] - beta_tab[:-1]) / (
        alpha_tab[1:] - alpha_tab[:-1] + 1e-12)
    c_seg = beta_tab[:-1] - s_seg * alpha_tab[:-1]
    pad1 = jnp.zeros(((-(n - 1)) % 8 + 8,), jnp.float32)
    s_seg = jnp.concatenate([s_seg, pad1])
    c_seg = jnp.concatenate([c_seg, pad1])

    scal = jnp.stack([
        jnp.full((LANES,), a0, jnp.float32),
        jnp.full((LANES,), a1, jnp.float32),
        jnp.full((LANES,), inv_w, jnp.float32),
        jnp.full((LANES,), beta_tab[-1], jnp.float32),
    ])

    tab_len = n + tab_pad
    seg_len = (n - 1) + (-(n - 1)) % 8 + 8
    mesh = plsc.VectorSubcoreMesh(core_axis_name="c", subcore_axis_name="s")
    run = pl.kernel(
        functools.partial(_sc_body, n, n_rows, n_cols),
        out_type=jax.ShapeDtypeStruct((n_rows, n_cols), jnp.float32),
        mesh=mesh,
        scratch_types=[
            pltpu.VMEM((M_BINS,), jnp.int32),
            pltpu.VMEM((tab_len,), jnp.float32),
            pltpu.VMEM((seg_len,), jnp.float32),
            pltpu.VMEM((seg_len,), jnp.float32),
            pltpu.VMEM((4, LANES), jnp.float32),
            pltpu.VMEM((SLAB, CCOLS), jnp.float32),
            pltpu.VMEM((SLAB, CCOLS), jnp.float32),
            pltpu.VMEM((SLAB, CCOLS), jnp.float32),
            pltpu.VMEM((SLAB, CCOLS), jnp.float32),
            pltpu.SemaphoreType.DMA,
            pltpu.SemaphoreType.DMA,
            pltpu.SemaphoreType.DMA,
            pltpu.SemaphoreType.DMA,
        ],
        compiler_params=pltpu.CompilerParams(
            needs_layout_passes=False, use_tc_tiling_on_sc=True),
    )
    return run(alpha, start, aext, s_seg, c_seg, scal)


# speedup vs baseline: 1.5493x; 1.1044x over previous
import functools

import jax
import jax.numpy as jnp
from jax import lax
from jax.experimental import pallas as pl
from jax.experimental.pallas import tpu as pltpu
from jax.experimental.pallas import tpu_sc as plsc

M_BINS = 65536
LANES = 16
SLAB = 8
CCOLS = 1024


def _sc_body(n_keys, n_rows, n_cols,
             alpha_hbm, start_hbm, aext_hbm, s_hbm, c_hbm, scal_hbm,
             out_hbm,
             start_v, aext_v, s_v, c_v, scal_v,
             in0_v, in1_v, out0_v, out1_v,
             sem_in0, sem_in1, sem_out0, sem_out1):
    nc = 2
    wid = lax.axis_index("s") * nc + lax.axis_index("c")
    rows_per_w = n_rows // 32
    base_row = wid * rows_per_w
    col_chunks = n_cols // CCOLS
    n_chunks = (rows_per_w // SLAB) * col_chunks
    in_bufs = (in0_v, in1_v)
    out_bufs = (out0_v, out1_v)
    sem_in = (sem_in0, sem_in1)
    sem_out = (sem_out0, sem_out1)

    def chunk_off(t):
        r0 = base_row + (t // col_chunks) * SLAB
        c0 = (t % col_chunks) * CCOLS
        return r0, c0

    pltpu.sync_copy(start_hbm, start_v)
    pltpu.sync_copy(aext_hbm, aext_v)
    pltpu.sync_copy(s_hbm, s_v)
    pltpu.sync_copy(c_hbm, c_v)
    pltpu.sync_copy(scal_hbm, scal_v)

    a0v = scal_v[0, :]
    a1v = scal_v[1, :]
    invwv = scal_v[2, :]
    b0v = scal_v[3, :]

    for b in range(2):
        r0, c0 = chunk_off(b)
        pltpu.async_copy(alpha_hbm.at[pl.ds(r0, SLAB), pl.ds(c0, CCOLS)],
                         in_bufs[b], sem_in[b])

    def compute(in_v, out_v):
        @plsc.parallel_loop(0, CCOLS, LANES, unroll=1)
        def vreg_body(o):
            for r in range(SLAB):
                v = in_v[r, pl.ds(o, LANES)]
                neg = v < 0.0
                vc = jnp.minimum(jnp.maximum(v, a0v), a1v)
                u = (vc - a0v) * invwv
                m = jnp.minimum(u.astype(jnp.int32), M_BINS - 1)
                j = plsc.load_gather(start_v, [m], mask=neg)
                g = plsc.load_gather(aext_v, [j], mask=neg)
                j = j + jnp.where(g <= vc, 1, 0)
                seg = jnp.minimum(j, n_keys - 1) - 1
                sv = plsc.load_gather(s_v, [seg], mask=neg)
                cv = plsc.load_gather(c_v, [seg], mask=neg)
                res = sv * vc + cv
                res = jnp.where(neg, res, b0v)
                out_v[r, pl.ds(o, LANES)] = res

    def pair_body(ii, _):
        for b in range(2):
            t = ii * 2 + b
            r0, c0 = chunk_off(t)
            pltpu.make_async_copy(
                alpha_hbm.at[pl.ds(r0, SLAB), pl.ds(c0, CCOLS)],
                in_bufs[b], sem_in[b]).wait()
            @pl.when(ii > 0)
            def _wait_out():
                pltpu.make_async_copy(
                    out_bufs[b],
                    out_hbm.at[pl.ds(r0, SLAB), pl.ds(c0, CCOLS)],
                    sem_out[b]).wait()
            compute(in_bufs[b], out_bufs[b])
            pltpu.async_copy(out_bufs[b],
                             out_hbm.at[pl.ds(r0, SLAB), pl.ds(c0, CCOLS)],
                             sem_out[b])
            @pl.when(t + 2 < n_chunks)
            def _prefetch():
                r2, c2 = chunk_off(t + 2)
                pltpu.async_copy(
                    alpha_hbm.at[pl.ds(r2, SLAB), pl.ds(c2, CCOLS)],
                    in_bufs[b], sem_in[b])
        return 0

    lax.fori_loop(0, n_chunks // 2, pair_body, 0)
    for b in range(2):
        r0, c0 = chunk_off(0)
        pltpu.make_async_copy(out_bufs[b],
                              out_hbm.at[pl.ds(r0, SLAB), pl.ds(c0, CCOLS)],
                              sem_out[b]).wait()


def kernel(alpha, alpha_tab, beta_tab):
    n = alpha_tab.shape[0]
    n_rows, n_cols = alpha.shape

    a0 = alpha_tab[0]
    a1 = alpha_tab[-1]
    inv_w = jnp.float32(M_BINS) / (a1 - a0)

    mk = jnp.minimum(((alpha_tab - a0) * inv_w).astype(jnp.int32), M_BINS - 1)
    counts = jnp.zeros((M_BINS,), jnp.int32).at[mk].add(1)
    start = (jnp.cumsum(counts) - counts).astype(jnp.int32)

    tab_pad = (-n) % 8 + 8
    aext = jnp.concatenate(
        [alpha_tab, jnp.full((tab_pad,), jnp.inf, jnp.float32)])
    s_seg = (beta_tab[1:] - beta_tab[:-1]) / (
        alpha_tab[1:] - alpha_tab[:-1] + 1e-12)
    c_seg = beta_tab[:-1] - s_seg * alpha_tab[:-1]
    pad1 = jnp.zeros(((-(n - 1)) % 8 + 8,), jnp.float32)
    s_seg = jnp.concatenate([s_seg, pad1])
    c_seg = jnp.concatenate([c_seg, pad1])

    scal = jnp.stack([
        jnp.full((LANES,), a0, jnp.float32),
        jnp.full((LANES,), a1, jnp.float32),
        jnp.full((LANES,), inv_w, jnp.float32),
        jnp.full((LANES,), beta_tab[-1], jnp.float32),
    ])

    tab_len = n + tab_pad
    seg_len = (n - 1) + (-(n - 1)) % 8 + 8
    mesh = plsc.VectorSubcoreMesh(core_axis_name="c", subcore_axis_name="s")
    run = pl.kernel(
        functools.partial(_sc_body, n, n_rows, n_cols),
        out_type=jax.ShapeDtypeStruct((n_rows, n_cols), jnp.float32),
        mesh=mesh,
        scratch_types=[
            pltpu.VMEM((M_BINS,), jnp.int32),
            pltpu.VMEM((tab_len,), jnp.float32),
            pltpu.VMEM((seg_len,), jnp.float32),
            pltpu.VMEM((seg_len,), jnp.float32),
            pltpu.VMEM((4, LANES), jnp.float32),
            pltpu.VMEM((SLAB, CCOLS), jnp.float32),
            pltpu.VMEM((SLAB, CCOLS), jnp.float32),
            pltpu.VMEM((SLAB, CCOLS), jnp.float32),
            pltpu.VMEM((SLAB, CCOLS), jnp.float32),
            pltpu.SemaphoreType.DMA,
            pltpu.SemaphoreType.DMA,
            pltpu.SemaphoreType.DMA,
            pltpu.SemaphoreType.DMA,
        ],
        compiler_params=pltpu.CompilerParams(
            needs_layout_passes=False, use_tc_tiling_on_sc=True),
    )
    return run(alpha, start, aext, s_seg, c_seg, scal)


# speedup vs baseline: 1.7302x; 1.1168x over previous
import functools

import jax
import jax.numpy as jnp
from jax import lax
from jax.experimental import pallas as pl
from jax.experimental.pallas import tpu as pltpu
from jax.experimental.pallas import tpu_sc as plsc

M_BINS = 65536
LANES = 16
SLAB = 8
CCOLS = 1024


def _sc_body(n_keys, n_rows, n_cols,
             alpha_hbm, start_hbm, aext_hbm, s_hbm, c_hbm, scal_hbm,
             out_hbm,
             start_v, aext_v, s_v, c_v, scal_v,
             in0_v, in1_v, out0_v, out1_v,
             sem_in0, sem_in1, sem_out0, sem_out1):
    nc = 2
    wid = lax.axis_index("s") * nc + lax.axis_index("c")
    rows_per_w = n_rows // 32
    base_row = wid * rows_per_w
    col_chunks = n_cols // CCOLS
    n_chunks = (rows_per_w // SLAB) * col_chunks
    in_bufs = (in0_v, in1_v)
    out_bufs = (out0_v, out1_v)
    sem_in = (sem_in0, sem_in1)
    sem_out = (sem_out0, sem_out1)

    def chunk_off(t):
        r0 = base_row + (t // col_chunks) * SLAB
        c0 = (t % col_chunks) * CCOLS
        return r0, c0

    pltpu.sync_copy(start_hbm, start_v)
    pltpu.sync_copy(aext_hbm, aext_v)
    pltpu.sync_copy(s_hbm, s_v)
    pltpu.sync_copy(c_hbm, c_v)
    pltpu.sync_copy(scal_hbm, scal_v)

    a0v = scal_v[0, :]
    a1v = scal_v[1, :]
    invwv = scal_v[2, :]
    b0v = scal_v[3, :]

    for b in range(2):
        r0, c0 = chunk_off(b)
        pltpu.async_copy(alpha_hbm.at[pl.ds(r0, SLAB), pl.ds(c0, CCOLS)],
                         in_bufs[b], sem_in[b])

    def compute(in_v, out_v):
        @plsc.parallel_loop(0, CCOLS, LANES, unroll=1)
        def vreg_body(o):
            for r in range(SLAB):
                v = in_v[r, pl.ds(o, LANES)]
                neg = v < 0.0
                vc = jnp.minimum(jnp.maximum(v, a0v), a1v)
                u = (vc - a0v) * invwv
                m = u.astype(jnp.int32)
                j = plsc.load_gather(start_v, [m], mask=neg)
                g = plsc.load_gather(aext_v, [j], mask=neg)
                j = j + (g <= vc).astype(jnp.int32)
                sv = plsc.load_gather(s_v, [j], mask=neg)
                cv = plsc.load_gather(c_v, [j], mask=neg)
                res = sv * vc + cv
                res = jnp.where(neg, res, b0v)
                out_v[r, pl.ds(o, LANES)] = res

    def pair_body(ii, _):
        for b in range(2):
            t = ii * 2 + b
            r0, c0 = chunk_off(t)
            pltpu.make_async_copy(
                alpha_hbm.at[pl.ds(r0, SLAB), pl.ds(c0, CCOLS)],
                in_bufs[b], sem_in[b]).wait()
            @pl.when(ii > 0)
            def _wait_out():
                pltpu.make_async_copy(
                    out_bufs[b],
                    out_hbm.at[pl.ds(r0, SLAB), pl.ds(c0, CCOLS)],
                    sem_out[b]).wait()
            compute(in_bufs[b], out_bufs[b])
            pltpu.async_copy(out_bufs[b],
                             out_hbm.at[pl.ds(r0, SLAB), pl.ds(c0, CCOLS)],
                             sem_out[b])
            @pl.when(t + 2 < n_chunks)
            def _prefetch():
                r2, c2 = chunk_off(t + 2)
                pltpu.async_copy(
                    alpha_hbm.at[pl.ds(r2, SLAB), pl.ds(c2, CCOLS)],
                    in_bufs[b], sem_in[b])
        return 0

    lax.fori_loop(0, n_chunks // 2, pair_body, 0)
    for b in range(2):
        r0, c0 = chunk_off(0)
        pltpu.make_async_copy(out_bufs[b],
                              out_hbm.at[pl.ds(r0, SLAB), pl.ds(c0, CCOLS)],
                              sem_out[b]).wait()


def kernel(alpha, alpha_tab, beta_tab):
    n = alpha_tab.shape[0]
    n_rows, n_cols = alpha.shape

    a0 = alpha_tab[0]
    a1 = alpha_tab[-1]
    inv_w = (jnp.float32(M_BINS) / (a1 - a0)) * jnp.float32(1.0 - 1e-6)

    mk = jnp.minimum(((alpha_tab - a0) * inv_w).astype(jnp.int32), M_BINS - 1)
    counts = jnp.zeros((M_BINS,), jnp.int32).at[mk].add(1)
    start = (jnp.cumsum(counts) - counts).astype(jnp.int32)

    tab_pad = (-n) % 8 + 8
    aext = jnp.concatenate(
        [alpha_tab, jnp.full((tab_pad,), jnp.inf, jnp.float32)])
    s_core = (beta_tab[1:] - beta_tab[:-1]) / (
        alpha_tab[1:] - alpha_tab[:-1] + 1e-12)
    c_core = beta_tab[:-1] - s_core * alpha_tab[:-1]
    pad1 = jnp.zeros(((-(n + 1)) % 8 + 8,), jnp.float32)
    s_seg = jnp.concatenate([jnp.zeros((1,), jnp.float32), s_core,
                             s_core[-1:], pad1])
    c_seg = jnp.concatenate([jnp.zeros((1,), jnp.float32), c_core,
                             c_core[-1:], pad1])

    scal = jnp.stack([
        jnp.full((LANES,), a0, jnp.float32),
        jnp.full((LANES,), a1, jnp.float32),
        jnp.full((LANES,), inv_w, jnp.float32),
        jnp.full((LANES,), beta_tab[-1], jnp.float32),
    ])

    tab_len = n + tab_pad
    seg_len = (n + 1) + (-(n + 1)) % 8 + 8
    mesh = plsc.VectorSubcoreMesh(core_axis_name="c", subcore_axis_name="s")
    run = pl.kernel(
        functools.partial(_sc_body, n, n_rows, n_cols),
        out_type=jax.ShapeDtypeStruct((n_rows, n_cols), jnp.float32),
        mesh=mesh,
        scratch_types=[
            pltpu.VMEM((M_BINS,), jnp.int32),
            pltpu.VMEM((tab_len,), jnp.float32),
            pltpu.VMEM((seg_len,), jnp.float32),
            pltpu.VMEM((seg_len,), jnp.float32),
            pltpu.VMEM((4, LANES), jnp.float32),
            pltpu.VMEM((SLAB, CCOLS), jnp.float32),
            pltpu.VMEM((SLAB, CCOLS), jnp.float32),
            pltpu.VMEM((SLAB, CCOLS), jnp.float32),
            pltpu.VMEM((SLAB, CCOLS), jnp.float32),
            pltpu.SemaphoreType.DMA,
            pltpu.SemaphoreType.DMA,
            pltpu.SemaphoreType.DMA,
            pltpu.SemaphoreType.DMA,
        ],
        compiler_params=pltpu.CompilerParams(
            needs_layout_passes=False, use_tc_tiling_on_sc=True),
    )
    return run(alpha, start, aext, s_seg, c_seg, scal)
